# baseline (speedup 1.0000x reference)
import jax
import jax.numpy as jnp
from jax.experimental import pallas as pl
from jax.experimental.pallas import tpu as pltpu

BATCH = 1024
EMB = 512
NUM_CLASSES = 100000
TILE_N = 4000


def _mm_kernel(x_ref, w_ref, o_ref):
    w = w_ref[...].astype(jnp.bfloat16)
    o_ref[...] = jax.lax.dot_general(
        w,
        x_ref[...],
        dimension_numbers=(((1,), (1,)), ((), ())),
        preferred_element_type=jnp.float32,
    )


def kernel(total_features, norm_weight):
    x = total_features.astype(jnp.bfloat16)
    grid = (pl.cdiv(NUM_CLASSES, TILE_N),)
    out_t = pl.pallas_call(
        _mm_kernel,
        grid=grid,
        in_specs=[
            pl.BlockSpec((BATCH, EMB), lambda i: (0, 0)),
            pl.BlockSpec((TILE_N, EMB), lambda i: (i, 0)),
        ],
        out_specs=pl.BlockSpec((TILE_N, BATCH), lambda i: (i, 0)),
        out_shape=jax.ShapeDtypeStruct((NUM_CLASSES, BATCH), jnp.float32),
        compiler_params=pltpu.CompilerParams(
            dimension_semantics=("parallel",),
        ),
    )(x, norm_weight)
    return out_t.T


# speedup vs baseline: 1.0125x; 1.0125x over previous
import jax
import jax.numpy as jnp
from jax.experimental import pallas as pl
from jax.experimental.pallas import tpu as pltpu

BATCH = 1024
EMB = 512
NUM_CLASSES = 100000
TILE_N = 4096


def _mm_kernel(x_ref, w_ref, o_ref):
    w = w_ref[...].astype(jnp.bfloat16)
    o_ref[...] = jax.lax.dot_general(
        w,
        x_ref[...].astype(jnp.bfloat16),
        dimension_numbers=(((1,), (1,)), ((), ())),
        preferred_element_type=jnp.float32,
    )


def kernel(total_features, norm_weight):
    x = total_features
    grid = (pl.cdiv(NUM_CLASSES, TILE_N),)
    out_t = pl.pallas_call(
        _mm_kernel,
        grid=grid,
        in_specs=[
            pl.BlockSpec((BATCH, EMB), lambda i: (0, 0)),
            pl.BlockSpec((TILE_N, EMB), lambda i: (i, 0)),
        ],
        out_specs=pl.BlockSpec((TILE_N, BATCH), lambda i: (i, 0)),
        out_shape=jax.ShapeDtypeStruct((NUM_CLASSES, BATCH), jnp.float32),
        compiler_params=pltpu.CompilerParams(
            dimension_semantics=("parallel",),
        ),
    )(x, norm_weight)
    return out_t.T
